# Initial kernel scaffold; baseline (speedup 1.0000x reference)
import jax
import jax.numpy as jnp
from jax.experimental import pallas as pl


def kernel(queries, query_batch_offsets, token_predicted_salience_score, query_spatial_indices, stacked_feature_maps, level_spatial_shapes, token_electron_scores, ln1_s, ln1_b, Wqkv, Wo, ln2_s, ln2_b, W1, b1, W2, b2):
    raise NotImplementedError("write your pallas kernel here")


# speedup vs baseline: 1.1225x; 1.1225x over previous
import functools

import jax
import jax.numpy as jnp
from jax import lax
from jax.experimental import pallas as pl
from jax.experimental.pallas import tpu as pltpu

B = 4
L = 20000
N = B * L
D = 256
H = 8
DH = D // H
FF = 1024
K = 1000
NF = DH // 4


def _attn_body(x_ref, py_ref, px_ref, inv_ref, ln1s_ref, ln1b_ref,
               wqkv_ref, wo_ref, o_ref):
    x = x_ref[0]
    mu = jnp.mean(x, axis=-1, keepdims=True)
    var = jnp.mean((x - mu) ** 2, axis=-1, keepdims=True)
    h = (x - mu) / jnp.sqrt(var + 1e-5) * ln1s_ref[...] + ln1b_ref[...]
    qkv = jnp.dot(h, wqkv_ref[...], preferred_element_type=jnp.float32)

    inv = inv_ref[...]
    py = py_ref[0]
    px = px_ref[0]
    ang = jnp.concatenate([py * inv, px * inv], axis=1)
    cos = jnp.cos(ang)
    sin = jnp.sin(ang)
    cs = jnp.concatenate([cos, cos], axis=1)
    sn = jnp.concatenate([-sin, sin], axis=1)

    def rope(t):
        partner = jnp.concatenate([t[:, DH // 2:], t[:, :DH // 2]], axis=1)
        return t * cs + partner * sn

    scale = 1.0 / (DH ** 0.5)
    outs = []
    for hh in range(H):
        qh = rope(qkv[:, hh * DH:(hh + 1) * DH])
        kh = rope(qkv[:, D + hh * DH:D + (hh + 1) * DH])
        vh = qkv[:, 2 * D + hh * DH:2 * D + (hh + 1) * DH]
        logits = lax.dot_general(
            qh, kh, (((1,), (1,)), ((), ())),
            preferred_element_type=jnp.float32) * scale
        m = jnp.max(logits, axis=-1, keepdims=True)
        p = jnp.exp(logits - m)
        p = p / jnp.sum(p, axis=-1, keepdims=True)
        outs.append(jnp.dot(p, vh, preferred_element_type=jnp.float32))
    o = jnp.concatenate(outs, axis=1)
    o_ref[0] = x + jnp.dot(o, wo_ref[...], preferred_element_type=jnp.float32)


def _ffn_body(x_ref, ln2s_ref, ln2b_ref, w1_ref, b1_ref, w2_ref, b2_ref,
              o_ref):
    x = x_ref[...]
    mu = jnp.mean(x, axis=-1, keepdims=True)
    var = jnp.mean((x - mu) ** 2, axis=-1, keepdims=True)
    h = (x - mu) / jnp.sqrt(var + 1e-5) * ln2s_ref[...] + ln2b_ref[...]
    a = jnp.dot(h, w1_ref[...], preferred_element_type=jnp.float32) + b1_ref[...]
    g = jax.nn.gelu(a)
    o_ref[...] = x + jnp.dot(g, w2_ref[...],
                             preferred_element_type=jnp.float32) + b2_ref[...]


def _attn_block(x_sel, posy, posx, inv, ln1_s, ln1_b, wqkv_p, wo):
    full = lambda s: pl.BlockSpec(s, lambda b: (0,) * len(s))
    return pl.pallas_call(
        _attn_body,
        grid=(B,),
        in_specs=[
            pl.BlockSpec((1, K, D), lambda b: (b, 0, 0)),
            pl.BlockSpec((1, K, 1), lambda b: (b, 0, 0)),
            pl.BlockSpec((1, K, 1), lambda b: (b, 0, 0)),
            full((1, NF)),
            full((D,)),
            full((D,)),
            full((D, 3 * D)),
            full((D, D)),
        ],
        out_specs=pl.BlockSpec((1, K, D), lambda b: (b, 0, 0)),
        out_shape=jax.ShapeDtypeStruct((B, K, D), jnp.float32),
    )(x_sel, posy, posx, inv, ln1_s, ln1_b, wqkv_p, wo)


FBLK = 2000


def _ffn_block(x, ln2_s, ln2_b, w1, b1, w2, b2):
    n = x.shape[0]
    assert n % FBLK == 0
    full = lambda s: pl.BlockSpec(s, lambda b: (0,) * len(s))
    return pl.pallas_call(
        _ffn_body,
        grid=(n // FBLK,),
        in_specs=[
            pl.BlockSpec((FBLK, D), lambda b: (b, 0)),
            full((D,)),
            full((D,)),
            full((D, FF)),
            full((FF,)),
            full((FF, D)),
            full((D,)),
        ],
        out_specs=pl.BlockSpec((FBLK, D), lambda b: (b, 0)),
        out_shape=jax.ShapeDtypeStruct((n, D), jnp.float32),
    )(x, ln2_s, ln2_b, w1, b1, w2, b2)


def kernel(queries, query_batch_offsets, token_predicted_salience_score,
           query_spatial_indices, stacked_feature_maps, level_spatial_shapes,
           token_electron_scores, ln1_s, ln1_b, Wqkv, Wo, ln2_s, ln2_b,
           W1, b1, W2, b2):
    del query_batch_offsets, stacked_feature_maps, level_spatial_shapes
    token_scores = token_electron_scores + token_predicted_salience_score
    _, idx = lax.top_k(token_scores.reshape(B, L), K)
    flat_idx = (idx + (jnp.arange(B, dtype=jnp.int32) * L)[:, None]).reshape(-1)

    x_sel = jnp.take(queries, flat_idx, axis=0).reshape(B, K, D)
    posy = jnp.take(query_spatial_indices[1], flat_idx).astype(jnp.float32)
    posx = jnp.take(query_spatial_indices[2], flat_idx).astype(jnp.float32)
    posy = posy.reshape(B, K, 1)
    posx = posx.reshape(B, K, 1)

    perm32 = jnp.concatenate([jnp.arange(0, DH, 2, dtype=jnp.int32),
                              jnp.arange(1, DH, 2, dtype=jnp.int32)])
    head_perm = (jnp.arange(2 * D, dtype=jnp.int32) // DH) * DH + \
        perm32[jnp.arange(2 * D, dtype=jnp.int32) % DH]
    col_perm = jnp.concatenate(
        [head_perm, jnp.arange(2 * D, 3 * D, dtype=jnp.int32)])
    wqkv_p = Wqkv[:, col_perm]
    inv = (1.0 / (100.0 ** (jnp.arange(NF, dtype=jnp.float32) / NF)))
    inv = inv.reshape(1, NF)

    sa_out = _attn_block(x_sel, posy, posx, inv, ln1_s, ln1_b, wqkv_p, Wo)

    q_upd = queries.at[flat_idx].set(sa_out.reshape(B * K, D))

    return _ffn_block(q_upd, ln2_s, ln2_b, W1, b1, W2, b2)
